# baseline (speedup 1.0000x reference)
import jax
import jax.numpy as jnp
import numpy as np
from jax.experimental import pallas as pl
from jax.experimental.pallas import tpu as pltpu

_B, _S, _N = 4, 16, 512
_DH = 32
_CB = 34 * _B


def _mm(a, b):
    return jax.lax.dot_general(a, b, (((1,), (0,)), ((), ())),
                               preferred_element_type=jnp.float32)


def _grinet_body(xs_ref, ms_ref, adj_ref, adjT_ref, embr_ref,
                 fWd, fbd, fWrz, fbrz, fWc, fbc, fWro, fbro, fWro2, fbro2,
                 bWd, bbd, bWrz, bbrz, bWc, bbc, bWro, bbro, bWro2, bbro2,
                 Wm1, bm1, Wm2, bm2,
                 out_ref,
                 fimp, bimp, fh, bh):
    adj = adj_ref[...]
    adjT = adjT_ref[...]
    s1 = adj / jnp.clip(jnp.sum(adj, axis=1, keepdims=True), 1e-8, None)
    s2 = adjT / jnp.clip(jnp.sum(adjT, axis=1, keepdims=True), 1e-8, None)
    s12 = jnp.concatenate([s1, s2], axis=0)
    s4 = jnp.concatenate([s12, _mm(s1, s1), _mm(s2, s2)], axis=0)

    fWd_ = fWd[...]; fbd_ = fbd[...]; bWd_ = bWd[...]; bbd_ = bbd[...]
    fWrz_ = fWrz[...]; fbrz_ = fbrz[...]; bWrz_ = bWrz[...]; bbrz_ = bbrz[...]
    fWc_ = fWc[...]; fbc_ = fbc[...]; bWc_ = bWc[...]; bbc_ = bbc[...]
    fWro_ = fWro[...]; fbro_ = fbro[...]; bWro_ = bWro[...]; bbro_ = bbro[...]
    fWro2_ = fWro2[...]; fbro2_ = fbro2[...]
    bWro2_ = bWro2[...]; bbro2_ = bbro2[...]

    HB = _DH * _B

    def step(i, carry):
        hf, hb = carry
        tf = i
        tb = _S - 1 - i
        xsf = xs_ref[tf]; msf = ms_ref[tf]
        xsb = xs_ref[tb]; msb = ms_ref[tb]
        omf = 1.0 - msf
        omb = 1.0 - msb
        hh = _mm(s4, jnp.concatenate([hf, hb], axis=1))
        h1 = hh[:_N]
        h2 = hh[_N:2 * _N]
        h11 = hh[2 * _N:3 * _N]
        h22 = hh[3 * _N:]
        x1f = _mm(hf, fWro_) + fbro_
        x1b = _mm(hb, bWro_) + bbro_
        xf1f = msf * xsf + omf * x1f
        xf1b = msb * xsb + omb * x1b
        smd = jnp.concatenate([xf1f, msf, xf1b, msb], axis=1)
        gd = _mm(s12, smd)
        d1 = gd[:_N]
        d2 = gd[_N:]
        dhf = jnp.maximum(_mm(jnp.concatenate(
            [xf1f, msf, d1[:, :2 * _B], d2[:, :2 * _B],
             hf, h1[:, :HB], h2[:, :HB]], axis=1), fWd_) + fbd_, 0.0)
        dhb = jnp.maximum(_mm(jnp.concatenate(
            [xf1b, msb, d1[:, 2 * _B:], d2[:, 2 * _B:],
             hb, h1[:, HB:], h2[:, HB:]], axis=1), bWd_) + bbd_, 0.0)
        x2f = _mm(dhf, fWro2_) + fbro2_
        x2b = _mm(dhb, bWro2_) + bbro2_
        xf2f = msf * xsf + omf * x2f
        xf2b = msb * xsb + omb * x2b
        smg = jnp.concatenate([xf2f, msf, xf2b, msb], axis=1)
        gg = _mm(s4, smg)
        g1 = gg[:_N]
        g2 = gg[_N:2 * _N]
        g11 = gg[2 * _N:3 * _N]
        g22 = gg[3 * _N:]
        rzf = jax.nn.sigmoid(_mm(jnp.concatenate(
            [xf2f, msf, g1[:, :2 * _B], g11[:, :2 * _B],
             g2[:, :2 * _B], g22[:, :2 * _B],
             hf, h1[:, :HB], h11[:, :HB], h2[:, :HB], h22[:, :HB]],
            axis=1), fWrz_) + fbrz_)
        rzb = jax.nn.sigmoid(_mm(jnp.concatenate(
            [xf2b, msb, g1[:, 2 * _B:], g11[:, 2 * _B:],
             g2[:, 2 * _B:], g22[:, 2 * _B:],
             hb, h1[:, HB:], h11[:, HB:], h2[:, HB:], h22[:, HB:]],
            axis=1), bWrz_) + bbrz_)
        rhf = rzf[:, :HB] * hf
        zf = rzf[:, HB:]
        rhb = rzb[:, :HB] * hb
        zb = rzb[:, HB:]
        ch = _mm(s4, jnp.concatenate([rhf, rhb], axis=1))
        c1 = ch[:_N]
        c2 = ch[_N:2 * _N]
        c11 = ch[2 * _N:3 * _N]
        c22 = ch[3 * _N:]
        cf = jnp.tanh(_mm(jnp.concatenate(
            [xf2f, msf, g1[:, :2 * _B], g11[:, :2 * _B],
             g2[:, :2 * _B], g22[:, :2 * _B],
             rhf, c1[:, :HB], c11[:, :HB], c2[:, :HB], c22[:, :HB]],
            axis=1), fWc_) + fbc_)
        cb = jnp.tanh(_mm(jnp.concatenate(
            [xf2b, msb, g1[:, 2 * _B:], g11[:, 2 * _B:],
             g2[:, 2 * _B:], g22[:, 2 * _B:],
             rhb, c1[:, HB:], c11[:, HB:], c2[:, HB:], c22[:, HB:]],
            axis=1), bWc_) + bbc_)
        hfn = zf * hf + (1.0 - zf) * cf
        hbn = zb * hb + (1.0 - zb) * cb
        fimp[tf] = x2f
        fh[tf] = hfn
        bimp[tb] = x2b
        bh[tb] = hbn
        return (hfn, hbn)

    h0 = jnp.zeros((_N, _DH * _B), jnp.float32)
    jax.lax.fori_loop(0, _S, step, (h0, h0))

    SN = _S * _N
    embr = jnp.broadcast_to(embr_ref[...][None], (_S, _N, 8 * _B))
    mi = jnp.concatenate([
        fimp[...].reshape(SN, _B),
        bimp[...].reshape(SN, _B),
        fh[...].reshape(SN, _DH * _B),
        bh[...].reshape(SN, _DH * _B),
        ms_ref[...].reshape(SN, _B),
        embr.reshape(SN, 8 * _B),
    ], axis=1)
    hmid = jnp.maximum(_mm(mi, Wm1[...]) + bm1[...], 0.0)
    o = _mm(hmid, Wm2[...]) + bm2[...]
    msa = ms_ref[...].reshape(SN, _B)
    xsa = xs_ref[...].reshape(SN, _B)
    out_ref[...] = jnp.where(msa > 0.5, xsa, o).reshape(_S, _N, _B)


def _kron(W):
    return jnp.kron(W, jnp.eye(_B, dtype=W.dtype))


def _rep(b):
    return jnp.repeat(b, _B)[None, :]


def _gperm(n_chunks):
    sm = [34 * j + k for j in range(n_chunks) for k in (0, 1)]
    hc = [34 * j + k for j in range(n_chunks) for k in range(2, 34)]
    return np.array(sm + hc)


_PD = _gperm(3)
_PG = _gperm(5)


@jax.jit
def kernel(x, edge_index, mask, adj, emb, params):
    del edge_index
    xs = jnp.transpose(x[..., 0], (1, 2, 0))
    ms = jnp.transpose(mask[..., 0].astype(jnp.float32), (1, 2, 0))
    adjT = adj.T
    embr = jnp.repeat(emb, _B, axis=1)

    def dirw(p):
        return [
            _kron(p['Wd'][_PD]), _rep(p['bd']),
            _kron(jnp.concatenate([p['Wr'], p['Wz']], axis=1)[_PG]),
            _rep(jnp.concatenate([p['br'], p['bz']])),
            _kron(p['Wc'][_PG]), _rep(p['bc']),
            _kron(p['Wro']), _rep(p['bro']),
            _kron(p['Wro2']), _rep(p['bro2']),
        ]

    fw = dirw(params['fwd'])
    bw = dirw(params['bwd'])
    Wm1 = _kron(params['Wm1'])
    bm1 = _rep(params['bm1'])
    Wm2 = _kron(params['Wm2'])
    bm2 = _rep(params['bm2'])

    out = pl.pallas_call(
        _grinet_body,
        out_shape=jax.ShapeDtypeStruct((_S, _N, _B), jnp.float32),
        scratch_shapes=[
            pltpu.VMEM((_S, _N, _B), jnp.float32),
            pltpu.VMEM((_S, _N, _B), jnp.float32),
            pltpu.VMEM((_S, _N, _DH * _B), jnp.float32),
            pltpu.VMEM((_S, _N, _DH * _B), jnp.float32),
        ],
    )(xs, ms, adj, adjT, embr, *fw, *bw, Wm1, bm1, Wm2, bm2)

    return jnp.transpose(out, (2, 0, 1))[..., None]


# speedup vs baseline: 2.0492x; 2.0492x over previous
import jax
import jax.numpy as jnp
from jax.experimental import pallas as pl
from jax.experimental.pallas import tpu as pltpu

_B, _S, _N = 4, 16, 512
_DH = 32
_CB = 34 * _B


def _mm(a, b):
    return jax.lax.dot_general(a, b, (((1,), (0,)), ((), ())),
                               preferred_element_type=jnp.float32)


def _mmT(a, b):
    return jax.lax.dot_general(a, b, (((0,), (0,)), ((), ())),
                               preferred_element_type=jnp.float32)


def _iota2(shape, dim):
    return jax.lax.broadcasted_iota(jnp.int32, shape, dim)


def _spread_rows(C):
    sh = (_B * C, C)
    return (_iota2(sh, 0) // _B == _iota2(sh, 1)).astype(jnp.float32)


def _spread_cols(K):
    sh = (K, _B * K)
    return (_iota2(sh, 1) // _B == _iota2(sh, 0)).astype(jnp.float32)


def _kron_in(W):
    C, K = W.shape
    spread = _mm(_mm(_spread_rows(C), W), _spread_cols(K))
    sh = (_B * C, _B * K)
    mask = (_iota2(sh, 0) % _B == _iota2(sh, 1) % _B).astype(jnp.float32)
    return spread * mask


def _bias_in(b):
    return _mm(b, _spread_cols(b.shape[1]))


def _grinet_body(xs_ref, ms_ref, adj_ref, emb_ref,
                 fWd_r, fbd_r, fWr_r, fbr_r, fWz_r, fbz_r, fWc_r, fbc_r,
                 fWro_r, fbro_r, fWro2_r, fbro2_r,
                 bWd_r, bbd_r, bWr_r, bbr_r, bWz_r, bbz_r, bWc_r, bbc_r,
                 bWro_r, bbro_r, bWro2_r, bbro2_r,
                 Wm1_r, bm1_r, Wm2_r, bm2_r,
                 out_ref,
                 fimp, bimp, fh, bh):
    adj = adj_ref[...]
    eye = (_iota2((_N, _N), 0) == _iota2((_N, _N), 1)).astype(jnp.float32)
    adjT = _mmT(adj, eye)
    s1 = adj / jnp.clip(jnp.sum(adj, axis=1, keepdims=True), 1e-8, None)
    s2 = adjT / jnp.clip(jnp.sum(adjT, axis=1, keepdims=True), 1e-8, None)
    s4 = jnp.concatenate([s1, s2, _mm(s1, s1), _mm(s2, s2)], axis=0)
    s12 = s4[:2 * _N]

    fWd_ = _kron_in(fWd_r[...]); fbd_ = _bias_in(fbd_r[...])
    bWd_ = _kron_in(bWd_r[...]); bbd_ = _bias_in(bbd_r[...])
    fWrz_ = _kron_in(jnp.concatenate([fWr_r[...], fWz_r[...]], axis=1))
    fbrz_ = _bias_in(jnp.concatenate([fbr_r[...], fbz_r[...]], axis=1))
    bWrz_ = _kron_in(jnp.concatenate([bWr_r[...], bWz_r[...]], axis=1))
    bbrz_ = _bias_in(jnp.concatenate([bbr_r[...], bbz_r[...]], axis=1))
    fWc_ = _kron_in(fWc_r[...]); fbc_ = _bias_in(fbc_r[...])
    bWc_ = _kron_in(bWc_r[...]); bbc_ = _bias_in(bbc_r[...])
    fWro_ = _kron_in(fWro_r[...]); fbro_ = _bias_in(fbro_r[...])
    bWro_ = _kron_in(bWro_r[...]); bbro_ = _bias_in(bbro_r[...])
    fWro2_ = _kron_in(fWro2_r[...]); fbro2_ = _bias_in(fbro2_r[...])
    bWro2_ = _kron_in(bWro2_r[...]); bbro2_ = _bias_in(bbro2_r[...])

    def split(m):
        return m[:, :_CB], m[:, _CB:]

    def step(i, carry):
        hf, hb = carry
        tf = i
        tb = _S - 1 - i
        xsf = xs_ref[tf]; msf = ms_ref[tf]
        xsb = xs_ref[tb]; msb = ms_ref[tb]
        omf = 1.0 - msf
        omb = 1.0 - msb
        x1f = _mm(hf, fWro_) + fbro_
        x1b = _mm(hb, bWro_) + bbro_
        xf1f = msf * xsf + omf * x1f
        xf1b = msb * xsb + omb * x1b
        Xd = jnp.concatenate([xf1f, msf, hf, xf1b, msb, hb], axis=1)
        g = _mm(s12, Xd)
        g1f, g1b = split(g[:_N])
        g2f, g2b = split(g[_N:])
        Xdf, Xdb = split(Xd)
        dhf = jnp.maximum(
            _mm(jnp.concatenate([Xdf, g1f, g2f], axis=1), fWd_) + fbd_, 0.0)
        dhb = jnp.maximum(
            _mm(jnp.concatenate([Xdb, g1b, g2b], axis=1), bWd_) + bbd_, 0.0)
        x2f = _mm(dhf, fWro2_) + fbro2_
        x2b = _mm(dhb, bWro2_) + bbro2_
        xf2f = msf * xsf + omf * x2f
        xf2b = msb * xsb + omb * x2b
        Xg = jnp.concatenate([xf2f, msf, hf, xf2b, msb, hb], axis=1)
        a = _mm(s4, Xg)
        a1f, a1b = split(a[:_N])
        a2f, a2b = split(a[_N:2 * _N])
        a11f, a11b = split(a[2 * _N:3 * _N])
        a22f, a22b = split(a[3 * _N:])
        Xgf, Xgb = split(Xg)
        rzf = jax.nn.sigmoid(
            _mm(jnp.concatenate([Xgf, a1f, a11f, a2f, a22f], axis=1), fWrz_)
            + fbrz_)
        rzb = jax.nn.sigmoid(
            _mm(jnp.concatenate([Xgb, a1b, a11b, a2b, a22b], axis=1), bWrz_)
            + bbrz_)
        rf = rzf[:, :_DH * _B]; zf = rzf[:, _DH * _B:]
        rb = rzb[:, :_DH * _B]; zb = rzb[:, _DH * _B:]
        Xc = jnp.concatenate([xf2f, msf, rf * hf, xf2b, msb, rb * hb], axis=1)
        ca = _mm(s4, Xc)
        c1f, c1b = split(ca[:_N])
        c2f, c2b = split(ca[_N:2 * _N])
        c11f, c11b = split(ca[2 * _N:3 * _N])
        c22f, c22b = split(ca[3 * _N:])
        Xcf, Xcb = split(Xc)
        cf = jnp.tanh(
            _mm(jnp.concatenate([Xcf, c1f, c11f, c2f, c22f], axis=1), fWc_)
            + fbc_)
        cb = jnp.tanh(
            _mm(jnp.concatenate([Xcb, c1b, c11b, c2b, c22b], axis=1), bWc_)
            + bbc_)
        hfn = zf * hf + (1.0 - zf) * cf
        hbn = zb * hb + (1.0 - zb) * cb
        fimp[tf] = x2f
        fh[tf] = hfn
        bimp[tb] = x2b
        bh[tb] = hbn
        return (hfn, hbn)

    h0 = jnp.zeros((_N, _DH * _B), jnp.float32)
    jax.lax.fori_loop(0, _S, step, (h0, h0))

    SN = _S * _N
    embr = _mm(emb_ref[...], _spread_cols(8))
    embb = jnp.broadcast_to(embr[None], (_S, _N, 8 * _B))
    Wm1_ = _kron_in(Wm1_r[...])
    bm1_ = _bias_in(bm1_r[...])
    Wm2_ = _kron_in(Wm2_r[...])
    bm2_ = _bias_in(bm2_r[...])
    mi = jnp.concatenate([
        fimp[...].reshape(SN, _B),
        bimp[...].reshape(SN, _B),
        fh[...].reshape(SN, _DH * _B),
        bh[...].reshape(SN, _DH * _B),
        ms_ref[...].reshape(SN, _B),
        embb.reshape(SN, 8 * _B),
    ], axis=1)
    hmid = jnp.maximum(_mm(mi, Wm1_) + bm1_, 0.0)
    o = _mm(hmid, Wm2_) + bm2_
    msa = ms_ref[...].reshape(SN, _B)
    xsa = xs_ref[...].reshape(SN, _B)
    out_ref[...] = jnp.where(msa > 0.5, xsa, o).reshape(_S, _N, _B)


@jax.jit
def kernel(x, edge_index, mask, adj, emb, params):
    del edge_index
    xs = jnp.transpose(x[..., 0], (1, 2, 0))
    ms = jnp.transpose(mask[..., 0].astype(jnp.float32), (1, 2, 0))

    def dirw(p):
        return [p['Wd'], p['bd'][None], p['Wr'], p['br'][None],
                p['Wz'], p['bz'][None], p['Wc'], p['bc'][None],
                p['Wro'], p['bro'][None], p['Wro2'], p['bro2'][None]]

    args = ([xs, ms, adj, emb] + dirw(params['fwd']) + dirw(params['bwd'])
            + [params['Wm1'], params['bm1'][None],
               params['Wm2'], params['bm2'][None]])

    out = pl.pallas_call(
        _grinet_body,
        out_shape=jax.ShapeDtypeStruct((_S, _N, _B), jnp.float32),
        scratch_shapes=[
            pltpu.VMEM((_S, _N, _B), jnp.float32),
            pltpu.VMEM((_S, _N, _B), jnp.float32),
            pltpu.VMEM((_S, _N, _DH * _B), jnp.float32),
            pltpu.VMEM((_S, _N, _DH * _B), jnp.float32),
        ],
    )(*args)

    return jnp.transpose(out, (2, 0, 1))[..., None]


# speedup vs baseline: 2.3953x; 1.1689x over previous
import jax
import jax.numpy as jnp
from jax.experimental import pallas as pl
from jax.experimental.pallas import tpu as pltpu

_B, _S, _N = 4, 16, 512
_DH = 32
_CB = 34 * _B


def _mm(a, b):
    return jax.lax.dot_general(a, b, (((1,), (0,)), ((), ())),
                               preferred_element_type=jnp.float32)


def _mmT(a, b):
    return jax.lax.dot_general(a, b, (((0,), (0,)), ((), ())),
                               preferred_element_type=jnp.float32)


def _iota2(shape, dim):
    return jax.lax.broadcasted_iota(jnp.int32, shape, dim)


def _spread_rows(C):
    sh = (_B * C, C)
    return (_iota2(sh, 0) // _B == _iota2(sh, 1)).astype(jnp.float32)


def _spread_cols(K):
    sh = (K, _B * K)
    return (_iota2(sh, 1) // _B == _iota2(sh, 0)).astype(jnp.float32)


def _kron_in(W):
    C, K = W.shape
    spread = _mm(_mm(_spread_rows(C), W), _spread_cols(K))
    sh = (_B * C, _B * K)
    mask = (_iota2(sh, 0) % _B == _iota2(sh, 1) % _B).astype(jnp.float32)
    return spread * mask


def _bias_in(b):
    return _mm(b, _spread_cols(b.shape[1]))


def _grinet_body(xq_ref, mq_ref, adj_ref, emb_ref,
                 fWd_r, fbd_r, fWr_r, fbr_r, fWz_r, fbz_r, fWc_r, fbc_r,
                 fWro_r, fbro_r, fWro2_r, fbro2_r,
                 bWd_r, bbd_r, bWr_r, bbr_r, bWz_r, bbz_r, bWc_r, bbc_r,
                 bWro_r, bbro_r, bWro2_r, bbro2_r,
                 Wm1_r, bm1_r, Wm2_r, bm2_r,
                 out_ref,
                 fimp, bimp, fh, bh, xs_ref, ms_ref):
    adj = adj_ref[...]
    eye = (_iota2((_N, _N), 0) == _iota2((_N, _N), 1)).astype(jnp.float32)
    adjT = _mmT(adj, eye)
    eye4 = (_iota2((_B, _B), 0) == _iota2((_B, _B), 1)).astype(jnp.float32)
    for t in range(_S):
        xs_ref[t] = _mmT(xq_ref[:, t, :], eye4)
        ms_ref[t] = _mmT(mq_ref[:, t, :], eye4)
    s1 = adj / jnp.clip(jnp.sum(adj, axis=1, keepdims=True), 1e-8, None)
    s2 = adjT / jnp.clip(jnp.sum(adjT, axis=1, keepdims=True), 1e-8, None)
    s4 = jnp.concatenate([s1, s2, _mm(s1, s1), _mm(s2, s2)], axis=0)
    s12 = s4[:2 * _N]

    fWd_ = _kron_in(fWd_r[...]); fbd_ = _bias_in(fbd_r[...][None, :])
    bWd_ = _kron_in(bWd_r[...]); bbd_ = _bias_in(bbd_r[...][None, :])
    fWrz_ = _kron_in(jnp.concatenate([fWr_r[...], fWz_r[...]], axis=1))
    fbrz_ = _bias_in(jnp.concatenate([fbr_r[...], fbz_r[...]])[None, :])
    bWrz_ = _kron_in(jnp.concatenate([bWr_r[...], bWz_r[...]], axis=1))
    bbrz_ = _bias_in(jnp.concatenate([bbr_r[...], bbz_r[...]])[None, :])
    fWc_ = _kron_in(fWc_r[...]); fbc_ = _bias_in(fbc_r[...][None, :])
    bWc_ = _kron_in(bWc_r[...]); bbc_ = _bias_in(bbc_r[...][None, :])
    fWro_ = _kron_in(fWro_r[...]); fbro_ = _bias_in(fbro_r[...][None, :])
    bWro_ = _kron_in(bWro_r[...]); bbro_ = _bias_in(bbro_r[...][None, :])
    fWro2_ = _kron_in(fWro2_r[...]); fbro2_ = _bias_in(fbro2_r[...][None, :])
    bWro2_ = _kron_in(bWro2_r[...]); bbro2_ = _bias_in(bbro2_r[...][None, :])

    def split(m):
        return m[:, :_CB], m[:, _CB:]

    def step(i, carry):
        hf, hb = carry
        tf = i
        tb = _S - 1 - i
        xsf = xs_ref[tf]; msf = ms_ref[tf]
        xsb = xs_ref[tb]; msb = ms_ref[tb]
        omf = 1.0 - msf
        omb = 1.0 - msb
        x1f = _mm(hf, fWro_) + fbro_
        x1b = _mm(hb, bWro_) + bbro_
        xf1f = msf * xsf + omf * x1f
        xf1b = msb * xsb + omb * x1b
        Xd = jnp.concatenate([xf1f, msf, hf, xf1b, msb, hb], axis=1)
        g = _mm(s12, Xd)
        g1f, g1b = split(g[:_N])
        g2f, g2b = split(g[_N:])
        Xdf, Xdb = split(Xd)
        dhf = jnp.maximum(
            _mm(jnp.concatenate([Xdf, g1f, g2f], axis=1), fWd_) + fbd_, 0.0)
        dhb = jnp.maximum(
            _mm(jnp.concatenate([Xdb, g1b, g2b], axis=1), bWd_) + bbd_, 0.0)
        x2f = _mm(dhf, fWro2_) + fbro2_
        x2b = _mm(dhb, bWro2_) + bbro2_
        xf2f = msf * xsf + omf * x2f
        xf2b = msb * xsb + omb * x2b
        Xg = jnp.concatenate([xf2f, msf, hf, xf2b, msb, hb], axis=1)
        a = _mm(s4, Xg)
        a1f, a1b = split(a[:_N])
        a2f, a2b = split(a[_N:2 * _N])
        a11f, a11b = split(a[2 * _N:3 * _N])
        a22f, a22b = split(a[3 * _N:])
        Xgf, Xgb = split(Xg)
        rzf = jax.nn.sigmoid(
            _mm(jnp.concatenate([Xgf, a1f, a11f, a2f, a22f], axis=1), fWrz_)
            + fbrz_)
        rzb = jax.nn.sigmoid(
            _mm(jnp.concatenate([Xgb, a1b, a11b, a2b, a22b], axis=1), bWrz_)
            + bbrz_)
        rf = rzf[:, :_DH * _B]; zf = rzf[:, _DH * _B:]
        rb = rzb[:, :_DH * _B]; zb = rzb[:, _DH * _B:]
        Xc = jnp.concatenate([xf2f, msf, rf * hf, xf2b, msb, rb * hb], axis=1)
        ca = _mm(s4, Xc)
        c1f, c1b = split(ca[:_N])
        c2f, c2b = split(ca[_N:2 * _N])
        c11f, c11b = split(ca[2 * _N:3 * _N])
        c22f, c22b = split(ca[3 * _N:])
        Xcf, Xcb = split(Xc)
        cf = jnp.tanh(
            _mm(jnp.concatenate([Xcf, c1f, c11f, c2f, c22f], axis=1), fWc_)
            + fbc_)
        cb = jnp.tanh(
            _mm(jnp.concatenate([Xcb, c1b, c11b, c2b, c22b], axis=1), bWc_)
            + bbc_)
        hfn = zf * hf + (1.0 - zf) * cf
        hbn = zb * hb + (1.0 - zb) * cb
        fimp[tf] = x2f
        fh[tf] = hfn
        bimp[tb] = x2b
        bh[tb] = hbn
        return (hfn, hbn)

    h0 = jnp.zeros((_N, _DH * _B), jnp.float32)
    jax.lax.fori_loop(0, _S, step, (h0, h0))

    SN = _S * _N
    embr = _mm(emb_ref[...], _spread_cols(8))
    embb = jnp.broadcast_to(embr[None], (_S, _N, 8 * _B))
    Wm1_ = _kron_in(Wm1_r[...])
    bm1_ = _bias_in(bm1_r[...][None, :])
    Wm2_ = _kron_in(Wm2_r[...])
    bm2_ = _bias_in(bm2_r[...][None, :])
    mi = jnp.concatenate([
        fimp[...].reshape(SN, _B),
        bimp[...].reshape(SN, _B),
        fh[...].reshape(SN, _DH * _B),
        bh[...].reshape(SN, _DH * _B),
        ms_ref[...].reshape(SN, _B),
        embb.reshape(SN, 8 * _B),
    ], axis=1)
    hmid = jnp.maximum(_mm(mi, Wm1_) + bm1_, 0.0)
    o = _mm(hmid, Wm2_) + bm2_
    msa = ms_ref[...].reshape(SN, _B)
    xsa = xs_ref[...].reshape(SN, _B)
    res = jnp.where(msa > 0.5, xsa, o)
    eyeN = (_iota2((_N, _N), 0) == _iota2((_N, _N), 1)).astype(jnp.float32)
    for t in range(_S):
        out_ref[:, t, :] = _mmT(res[t * _N:(t + 1) * _N], eyeN)


@jax.jit
def kernel(x, edge_index, mask, adj, emb, params):
    del edge_index
    xq = x[..., 0]
    mq = mask[..., 0].astype(jnp.float32)

    def dirw(p):
        return [p['Wd'], p['bd'], p['Wr'], p['br'],
                p['Wz'], p['bz'], p['Wc'], p['bc'],
                p['Wro'], p['bro'], p['Wro2'], p['bro2']]

    args = ([xq, mq, adj, emb] + dirw(params['fwd']) + dirw(params['bwd'])
            + [params['Wm1'], params['bm1'], params['Wm2'], params['bm2']])

    out = pl.pallas_call(
        _grinet_body,
        out_shape=jax.ShapeDtypeStruct((_B, _S, _N), jnp.float32),
        scratch_shapes=[
            pltpu.VMEM((_S, _N, _B), jnp.float32),
            pltpu.VMEM((_S, _N, _B), jnp.float32),
            pltpu.VMEM((_S, _N, _DH * _B), jnp.float32),
            pltpu.VMEM((_S, _N, _DH * _B), jnp.float32),
            pltpu.VMEM((_S, _N, _B), jnp.float32),
            pltpu.VMEM((_S, _N, _B), jnp.float32),
        ],
    )(*args)

    return out[..., None]


# speedup vs baseline: 2.4269x; 1.0132x over previous
import jax
import jax.numpy as jnp
from jax.experimental import pallas as pl
from jax.experimental.pallas import tpu as pltpu

_B, _S, _N = 4, 16, 512
_DH = 32
_CB = 34 * _B


def _mm(a, b):
    return jax.lax.dot_general(a, b, (((1,), (0,)), ((), ())),
                               preferred_element_type=jnp.float32)


def _mmT(a, b):
    return jax.lax.dot_general(a, b, (((0,), (0,)), ((), ())),
                               preferred_element_type=jnp.float32)


def _iota2(shape, dim):
    return jax.lax.broadcasted_iota(jnp.int32, shape, dim)


def _spread_rows(C):
    sh = (_B * C, C)
    return (_iota2(sh, 0) // _B == _iota2(sh, 1)).astype(jnp.float32)


def _spread_cols(K):
    sh = (K, _B * K)
    return (_iota2(sh, 1) // _B == _iota2(sh, 0)).astype(jnp.float32)


def _kron_in(W):
    C, K = W.shape
    spread = _mm(_mm(_spread_rows(C), W), _spread_cols(K))
    sh = (_B * C, _B * K)
    mask = (_iota2(sh, 0) % _B == _iota2(sh, 1) % _B).astype(jnp.float32)
    return spread * mask


def _bias_in(b):
    return _mm(b, _spread_cols(b.shape[1]))


def _grinet_body(xq_ref, mq_ref, adj_ref, emb_ref,
                 fWd_r, fbd_r, fWr_r, fbr_r, fWz_r, fbz_r, fWc_r, fbc_r,
                 fWro_r, fbro_r, fWro2_r, fbro2_r,
                 bWd_r, bbd_r, bWr_r, bbr_r, bWz_r, bbz_r, bWc_r, bbc_r,
                 bWro_r, bbro_r, bWro2_r, bbro2_r,
                 Wm1_r, bm1_r, Wm2_r, bm2_r,
                 out_ref,
                 fimp, bimp, fh, bh, xs_ref, ms_ref):
    adj = adj_ref[...]
    eye = (_iota2((_N, _N), 0) == _iota2((_N, _N), 1)).astype(jnp.float32)
    adjT = _mmT(adj, eye)
    eye4 = (_iota2((_B, _B), 0) == _iota2((_B, _B), 1)).astype(jnp.float32)
    for t in range(_S):
        xs_ref[t] = _mmT(xq_ref[:, t, :], eye4)
        ms_ref[t] = _mmT(mq_ref[:, t, :], eye4)
    s1 = adj / jnp.clip(jnp.sum(adj, axis=1, keepdims=True), 1e-8, None)
    s2 = adjT / jnp.clip(jnp.sum(adjT, axis=1, keepdims=True), 1e-8, None)
    s4 = jnp.concatenate([s1, s2, _mm(s1, s1), _mm(s2, s2)], axis=0)
    s12 = s4[:2 * _N]

    fWd_ = _kron_in(fWd_r[...]); fbd_ = _bias_in(fbd_r[...][None, :])
    bWd_ = _kron_in(bWd_r[...]); bbd_ = _bias_in(bbd_r[...][None, :])
    fWrz_ = _kron_in(jnp.concatenate([fWr_r[...], fWz_r[...]], axis=1))
    fbrz_ = _bias_in(jnp.concatenate([fbr_r[...], fbz_r[...]])[None, :])
    bWrz_ = _kron_in(jnp.concatenate([bWr_r[...], bWz_r[...]], axis=1))
    bbrz_ = _bias_in(jnp.concatenate([bbr_r[...], bbz_r[...]])[None, :])
    fWc_ = _kron_in(fWc_r[...]); fbc_ = _bias_in(fbc_r[...][None, :])
    bWc_ = _kron_in(bWc_r[...]); bbc_ = _bias_in(bbc_r[...][None, :])
    fWro_ = _kron_in(fWro_r[...]); fbro_ = _bias_in(fbro_r[...][None, :])
    bWro_ = _kron_in(bWro_r[...]); bbro_ = _bias_in(bbro_r[...][None, :])
    fWro2_ = _kron_in(fWro2_r[...]); fbro2_ = _bias_in(fbro2_r[...][None, :])
    bWro2_ = _kron_in(bWro2_r[...]); bbro2_ = _bias_in(bbro2_r[...][None, :])

    def split(m):
        return m[:, :_CB], m[:, _CB:]

    def step(i, carry):
        hf, hb = carry
        tf = i
        tb = _S - 1 - i
        xsf = xs_ref[tf]; msf = ms_ref[tf]
        xsb = xs_ref[tb]; msb = ms_ref[tb]
        omf = 1.0 - msf
        omb = 1.0 - msb
        x1f = _mm(hf, fWro_) + fbro_
        x1b = _mm(hb, bWro_) + bbro_
        xf1f = msf * xsf + omf * x1f
        xf1b = msb * xsb + omb * x1b
        Xd = jnp.concatenate([xf1f, msf, hf, xf1b, msb, hb], axis=1)
        g = _mm(s12, Xd)
        g1f, g1b = split(g[:_N])
        g2f, g2b = split(g[_N:])
        Xdf, Xdb = split(Xd)
        dhf = jnp.maximum(
            _mm(jnp.concatenate([Xdf, g1f, g2f], axis=1), fWd_) + fbd_, 0.0)
        dhb = jnp.maximum(
            _mm(jnp.concatenate([Xdb, g1b, g2b], axis=1), bWd_) + bbd_, 0.0)
        x2f = _mm(dhf, fWro2_) + fbro2_
        x2b = _mm(dhb, bWro2_) + bbro2_
        xf2f = msf * xsf + omf * x2f
        xf2b = msb * xsb + omb * x2b
        Xg = jnp.concatenate([xf2f, msf, hf, xf2b, msb, hb], axis=1)
        a = _mm(s4, Xg)
        a1f, a1b = split(a[:_N])
        a2f, a2b = split(a[_N:2 * _N])
        a11f, a11b = split(a[2 * _N:3 * _N])
        a22f, a22b = split(a[3 * _N:])
        Xgf, Xgb = split(Xg)
        rzf = jax.nn.sigmoid(
            _mm(jnp.concatenate([Xgf, a1f, a11f, a2f, a22f], axis=1), fWrz_)
            + fbrz_)
        rzb = jax.nn.sigmoid(
            _mm(jnp.concatenate([Xgb, a1b, a11b, a2b, a22b], axis=1), bWrz_)
            + bbrz_)
        rf = rzf[:, :_DH * _B]; zf = rzf[:, _DH * _B:]
        rb = rzb[:, :_DH * _B]; zb = rzb[:, _DH * _B:]
        Xc = jnp.concatenate([xf2f, msf, rf * hf, xf2b, msb, rb * hb], axis=1)
        ca = _mm(s4, Xc)
        c1f, c1b = split(ca[:_N])
        c2f, c2b = split(ca[_N:2 * _N])
        c11f, c11b = split(ca[2 * _N:3 * _N])
        c22f, c22b = split(ca[3 * _N:])
        Xcf, Xcb = split(Xc)
        cf = jnp.tanh(
            _mm(jnp.concatenate([Xcf, c1f, c11f, c2f, c22f], axis=1), fWc_)
            + fbc_)
        cb = jnp.tanh(
            _mm(jnp.concatenate([Xcb, c1b, c11b, c2b, c22b], axis=1), bWc_)
            + bbc_)
        hfn = zf * hf + (1.0 - zf) * cf
        hbn = zb * hb + (1.0 - zb) * cb
        fimp[tf] = x2f
        fh[tf] = hfn
        bimp[tb] = x2b
        bh[tb] = hbn
        return (hfn, hbn)

    h0 = jnp.zeros((_N, _DH * _B), jnp.float32)
    jax.lax.fori_loop(0, _S // 2, lambda i, c: step(2 * i + 1, step(2 * i, c)), (h0, h0))

    SN = _S * _N
    embr = _mm(emb_ref[...], _spread_cols(8))
    embb = jnp.broadcast_to(embr[None], (_S, _N, 8 * _B))
    Wm1_ = _kron_in(Wm1_r[...])
    bm1_ = _bias_in(bm1_r[...][None, :])
    Wm2_ = _kron_in(Wm2_r[...])
    bm2_ = _bias_in(bm2_r[...][None, :])
    mi = jnp.concatenate([
        fimp[...].reshape(SN, _B),
        bimp[...].reshape(SN, _B),
        fh[...].reshape(SN, _DH * _B),
        bh[...].reshape(SN, _DH * _B),
        ms_ref[...].reshape(SN, _B),
        embb.reshape(SN, 8 * _B),
    ], axis=1)
    hmid = jnp.maximum(_mm(mi, Wm1_) + bm1_, 0.0)
    o = _mm(hmid, Wm2_) + bm2_
    msa = ms_ref[...].reshape(SN, _B)
    xsa = xs_ref[...].reshape(SN, _B)
    res = jnp.where(msa > 0.5, xsa, o)
    eyeN = (_iota2((_N, _N), 0) == _iota2((_N, _N), 1)).astype(jnp.float32)
    for t in range(_S):
        out_ref[:, t, :] = _mmT(res[t * _N:(t + 1) * _N], eyeN)


@jax.jit
def kernel(x, edge_index, mask, adj, emb, params):
    del edge_index
    xq = x[..., 0]
    mq = mask[..., 0].astype(jnp.float32)

    def dirw(p):
        return [p['Wd'], p['bd'], p['Wr'], p['br'],
                p['Wz'], p['bz'], p['Wc'], p['bc'],
                p['Wro'], p['bro'], p['Wro2'], p['bro2']]

    args = ([xq, mq, adj, emb] + dirw(params['fwd']) + dirw(params['bwd'])
            + [params['Wm1'], params['bm1'], params['Wm2'], params['bm2']])

    out = pl.pallas_call(
        _grinet_body,
        out_shape=jax.ShapeDtypeStruct((_B, _S, _N), jnp.float32),
        scratch_shapes=[
            pltpu.VMEM((_S, _N, _B), jnp.float32),
            pltpu.VMEM((_S, _N, _B), jnp.float32),
            pltpu.VMEM((_S, _N, _DH * _B), jnp.float32),
            pltpu.VMEM((_S, _N, _DH * _B), jnp.float32),
            pltpu.VMEM((_S, _N, _B), jnp.float32),
            pltpu.VMEM((_S, _N, _B), jnp.float32),
        ],
    )(*args)

    return out[..., None]


# speedup vs baseline: 2.4563x; 1.0121x over previous
import jax
import jax.numpy as jnp
from jax.experimental import pallas as pl
from jax.experimental.pallas import tpu as pltpu

_B, _S, _N = 4, 16, 512
_DH = 32
_CB = 34 * _B


def _mm(a, b):
    return jax.lax.dot_general(a, b, (((1,), (0,)), ((), ())),
                               preferred_element_type=jnp.float32)


def _mmT(a, b):
    return jax.lax.dot_general(a, b, (((0,), (0,)), ((), ())),
                               preferred_element_type=jnp.float32)


def _iota2(shape, dim):
    return jax.lax.broadcasted_iota(jnp.int32, shape, dim)


def _spread_rows(C):
    sh = (_B * C, C)
    return (_iota2(sh, 0) // _B == _iota2(sh, 1)).astype(jnp.float32)


def _spread_cols(K):
    sh = (K, _B * K)
    return (_iota2(sh, 1) // _B == _iota2(sh, 0)).astype(jnp.float32)


def _kron_in(W):
    C, K = W.shape
    spread = _mm(_mm(_spread_rows(C), W), _spread_cols(K))
    sh = (_B * C, _B * K)
    mask = (_iota2(sh, 0) % _B == _iota2(sh, 1) % _B).astype(jnp.float32)
    return spread * mask


def _bias_in(b):
    return _mm(b, _spread_cols(b.shape[1]))


def _grinet_body(xq_ref, mq_ref, adj_ref, emb_ref,
                 fWd_r, fbd_r, fWr_r, fbr_r, fWz_r, fbz_r, fWc_r, fbc_r,
                 fWro_r, fbro_r, fWro2_r, fbro2_r,
                 bWd_r, bbd_r, bWr_r, bbr_r, bWz_r, bbz_r, bWc_r, bbc_r,
                 bWro_r, bbro_r, bWro2_r, bbro2_r,
                 Wm1_r, bm1_r, Wm2_r, bm2_r,
                 out_ref,
                 fimp, bimp, fh, bh, xs_ref, ms_ref):
    adj = adj_ref[...]
    eye = (_iota2((_N, _N), 0) == _iota2((_N, _N), 1)).astype(jnp.float32)
    adjT = _mmT(adj, eye)
    eye4 = (_iota2((_B, _B), 0) == _iota2((_B, _B), 1)).astype(jnp.float32)
    for t in range(_S):
        xs_ref[t] = _mmT(xq_ref[:, t, :], eye4)
        ms_ref[t] = _mmT(mq_ref[:, t, :], eye4)
    s1 = adj / jnp.clip(jnp.sum(adj, axis=1, keepdims=True), 1e-8, None)
    s2 = adjT / jnp.clip(jnp.sum(adjT, axis=1, keepdims=True), 1e-8, None)
    s4 = jnp.concatenate([s1, s2, _mm(s1, s1), _mm(s2, s2)], axis=0)
    s12 = s4[:2 * _N]

    fWd_ = _kron_in(fWd_r[...]); fbd_ = _bias_in(fbd_r[...][None, :])
    bWd_ = _kron_in(bWd_r[...]); bbd_ = _bias_in(bbd_r[...][None, :])
    fWrz_ = _kron_in(jnp.concatenate([fWr_r[...], fWz_r[...]], axis=1))
    fbrz_ = _bias_in(jnp.concatenate([fbr_r[...], fbz_r[...]])[None, :])
    bWrz_ = _kron_in(jnp.concatenate([bWr_r[...], bWz_r[...]], axis=1))
    bbrz_ = _bias_in(jnp.concatenate([bbr_r[...], bbz_r[...]])[None, :])
    fWc_ = _kron_in(fWc_r[...]); fbc_ = _bias_in(fbc_r[...][None, :])
    bWc_ = _kron_in(bWc_r[...]); bbc_ = _bias_in(bbc_r[...][None, :])
    fWro_ = _kron_in(fWro_r[...]); fbro_ = _bias_in(fbro_r[...][None, :])
    bWro_ = _kron_in(bWro_r[...]); bbro_ = _bias_in(bbro_r[...][None, :])
    fWro2_ = _kron_in(fWro2_r[...]); fbro2_ = _bias_in(fbro2_r[...][None, :])
    bWro2_ = _kron_in(bWro2_r[...]); bbro2_ = _bias_in(bbro2_r[...][None, :])

    def split(m):
        return m[:, :_CB], m[:, _CB:]

    def step(i, carry):
        hf, hb = carry
        tf = i
        tb = _S - 1 - i
        xsf = xs_ref[tf]; msf = ms_ref[tf]
        xsb = xs_ref[tb]; msb = ms_ref[tb]
        omf = 1.0 - msf
        omb = 1.0 - msb
        x1f = _mm(hf, fWro_) + fbro_
        x1b = _mm(hb, bWro_) + bbro_
        xf1f = msf * xsf + omf * x1f
        xf1b = msb * xsb + omb * x1b
        Xd = jnp.concatenate([xf1f, msf, hf, xf1b, msb, hb], axis=1)
        g = _mm(s12, Xd)
        g1f, g1b = split(g[:_N])
        g2f, g2b = split(g[_N:])
        Xdf, Xdb = split(Xd)
        dhf = jnp.maximum(
            _mm(jnp.concatenate([Xdf, g1f, g2f], axis=1), fWd_) + fbd_, 0.0)
        dhb = jnp.maximum(
            _mm(jnp.concatenate([Xdb, g1b, g2b], axis=1), bWd_) + bbd_, 0.0)
        x2f = _mm(dhf, fWro2_) + fbro2_
        x2b = _mm(dhb, bWro2_) + bbro2_
        xf2f = msf * xsf + omf * x2f
        xf2b = msb * xsb + omb * x2b
        Xg = jnp.concatenate([xf2f, msf, hf, xf2b, msb, hb], axis=1)
        a = _mm(s4, Xg)
        a1f, a1b = split(a[:_N])
        a2f, a2b = split(a[_N:2 * _N])
        a11f, a11b = split(a[2 * _N:3 * _N])
        a22f, a22b = split(a[3 * _N:])
        Xgf, Xgb = split(Xg)
        rzf = jax.nn.sigmoid(
            _mm(jnp.concatenate([Xgf, a1f, a11f, a2f, a22f], axis=1), fWrz_)
            + fbrz_)
        rzb = jax.nn.sigmoid(
            _mm(jnp.concatenate([Xgb, a1b, a11b, a2b, a22b], axis=1), bWrz_)
            + bbrz_)
        rf = rzf[:, :_DH * _B]; zf = rzf[:, _DH * _B:]
        rb = rzb[:, :_DH * _B]; zb = rzb[:, _DH * _B:]
        Xc = jnp.concatenate([xf2f, msf, rf * hf, xf2b, msb, rb * hb], axis=1)
        ca = _mm(s4, Xc)
        c1f, c1b = split(ca[:_N])
        c2f, c2b = split(ca[_N:2 * _N])
        c11f, c11b = split(ca[2 * _N:3 * _N])
        c22f, c22b = split(ca[3 * _N:])
        Xcf, Xcb = split(Xc)
        cf = jnp.tanh(
            _mm(jnp.concatenate([Xcf, c1f, c11f, c2f, c22f], axis=1), fWc_)
            + fbc_)
        cb = jnp.tanh(
            _mm(jnp.concatenate([Xcb, c1b, c11b, c2b, c22b], axis=1), bWc_)
            + bbc_)
        hfn = zf * hf + (1.0 - zf) * cf
        hbn = zb * hb + (1.0 - zb) * cb
        fimp[tf] = x2f
        fh[tf] = hfn
        bimp[tb] = x2b
        bh[tb] = hbn
        return (hfn, hbn)

    h0 = jnp.zeros((_N, _DH * _B), jnp.float32)
    jax.lax.fori_loop(0, _S // 4, lambda i, c: step(4 * i + 3, step(4 * i + 2, step(4 * i + 1, step(4 * i, c)))), (h0, h0))

    SN = _S * _N
    embr = _mm(emb_ref[...], _spread_cols(8))
    embb = jnp.broadcast_to(embr[None], (_S, _N, 8 * _B))
    Wm1_ = _kron_in(Wm1_r[...])
    bm1_ = _bias_in(bm1_r[...][None, :])
    Wm2_ = _kron_in(Wm2_r[...])
    bm2_ = _bias_in(bm2_r[...][None, :])
    mi = jnp.concatenate([
        fimp[...].reshape(SN, _B),
        bimp[...].reshape(SN, _B),
        fh[...].reshape(SN, _DH * _B),
        bh[...].reshape(SN, _DH * _B),
        ms_ref[...].reshape(SN, _B),
        embb.reshape(SN, 8 * _B),
    ], axis=1)
    hmid = jnp.maximum(_mm(mi, Wm1_) + bm1_, 0.0)
    o = _mm(hmid, Wm2_) + bm2_
    msa = ms_ref[...].reshape(SN, _B)
    xsa = xs_ref[...].reshape(SN, _B)
    res = jnp.where(msa > 0.5, xsa, o)
    eyeN = (_iota2((_N, _N), 0) == _iota2((_N, _N), 1)).astype(jnp.float32)
    for t in range(_S):
        out_ref[:, t, :] = _mmT(res[t * _N:(t + 1) * _N], eyeN)


@jax.jit
def kernel(x, edge_index, mask, adj, emb, params):
    del edge_index
    xq = x[..., 0]
    mq = mask[..., 0].astype(jnp.float32)

    def dirw(p):
        return [p['Wd'], p['bd'], p['Wr'], p['br'],
                p['Wz'], p['bz'], p['Wc'], p['bc'],
                p['Wro'], p['bro'], p['Wro2'], p['bro2']]

    args = ([xq, mq, adj, emb] + dirw(params['fwd']) + dirw(params['bwd'])
            + [params['Wm1'], params['bm1'], params['Wm2'], params['bm2']])

    out = pl.pallas_call(
        _grinet_body,
        out_shape=jax.ShapeDtypeStruct((_B, _S, _N), jnp.float32),
        scratch_shapes=[
            pltpu.VMEM((_S, _N, _B), jnp.float32),
            pltpu.VMEM((_S, _N, _B), jnp.float32),
            pltpu.VMEM((_S, _N, _DH * _B), jnp.float32),
            pltpu.VMEM((_S, _N, _DH * _B), jnp.float32),
            pltpu.VMEM((_S, _N, _B), jnp.float32),
            pltpu.VMEM((_S, _N, _B), jnp.float32),
        ],
    )(*args)

    return out[..., None]
